# baseline (speedup 1.0000x reference)
import functools

import jax
import jax.numpy as jnp
from jax import lax
from jax.experimental import pallas as pl
from jax.experimental.pallas import tpu as pltpu
from jax.experimental.pallas import tpu_sc as plsc

T = 2048
H = 2048
E = 16
I = 1408
ISH = 2816
NG = 4
TKG = 2
TOPK = 2
RSF = 2.5

P = T * TOPK
BM = 256
NT = P // BM + E
PT = NT * BM

BN_SH = 1408
BH = 1024


NC_SC = 2
NS_SC = 16
NW_SC = NC_SC * NS_SC
GC = 32


def _sc_row_gather(table, idx, n_rows, used_rows=None):
    D = table.shape[1]
    per_w = n_rows // NW_SC
    n_chunks = per_w // GC
    idx2d = idx.reshape(-1, GC)
    mesh = plsc.VectorSubcoreMesh(core_axis_name="c", subcore_axis_name="s")

    @functools.partial(
        pl.kernel, mesh=mesh,
        out_type=jax.ShapeDtypeStruct((n_rows, D), table.dtype),
        scratch_types=[
            pltpu.VMEM((n_chunks, GC), jnp.int32),
            pltpu.VMEM((2, GC, D), table.dtype),
            pltpu.SemaphoreType.DMA,
            pltpu.SemaphoreType.DMA,
        ],
    )
    def gather_k(table_hbm, idx_hbm, out_hbm, idx_v, rows_v, sem0, sem1):
        wid = lax.axis_index("s") * NC_SC + lax.axis_index("c")
        base = wid * per_w
        pltpu.sync_copy(idx_hbm.at[pl.ds(wid * n_chunks, n_chunks)], idx_v)
        pltpu.async_copy(table_hbm.at[idx_v.at[0]], rows_v.at[0], sem0)

        def body(g, carry):
            i0 = g * 2
            pltpu.make_async_copy(
                table_hbm.at[idx_v.at[i0]], rows_v.at[0], sem0).wait()
            pltpu.async_copy(
                table_hbm.at[idx_v.at[i0 + 1]], rows_v.at[1], sem1)
            pltpu.sync_copy(
                rows_v.at[0],
                out_hbm.at[pl.ds(pl.multiple_of(base + i0 * GC, GC), GC)])
            pltpu.make_async_copy(
                table_hbm.at[idx_v.at[i0 + 1]], rows_v.at[1], sem1).wait()

            @pl.when(i0 + 2 < n_chunks)
            def _():
                pltpu.async_copy(
                    table_hbm.at[idx_v.at[i0 + 2]], rows_v.at[0], sem0)

            pltpu.sync_copy(
                rows_v.at[1],
                out_hbm.at[pl.ds(pl.multiple_of(base + (i0 + 1) * GC, GC),
                                 GC)])
            return carry

        lax.fori_loop(0, n_chunks // 2, body, 0)

    return gather_k(table, idx2d)


def _gu_body(te_ref, nt_ref, x_ref, wg_ref, wu_ref, act_ref):
    t = pl.program_id(0)

    @pl.when(t < nt_ref[0])
    def _():
        xb = x_ref[...].astype(jnp.float32)
        g = jnp.dot(xb, wg_ref[0], preferred_element_type=jnp.float32)
        u = jnp.dot(xb, wu_ref[0], preferred_element_type=jnp.float32)
        act_ref[...] = (g * jax.nn.sigmoid(g) * u).astype(jnp.bfloat16)


def _grouped_gate_up(x_pad, w_gate_up, tile_expert, n_tiles):
    def xmap(t, te, nt):
        return jnp.where(t < nt[0], t, nt[0] - 1), 0

    def wmap_g(t, te, nt):
        return te[jnp.where(t < nt[0], t, nt[0] - 1)], 0, 0

    def wmap_u(t, te, nt):
        return te[jnp.where(t < nt[0], t, nt[0] - 1)], 0, 1

    def omap(t, te, nt):
        return jnp.where(t < nt[0], t, nt[0] - 1), 0

    grid_spec = pltpu.PrefetchScalarGridSpec(
        num_scalar_prefetch=2,
        grid=(NT,),
        in_specs=[
            pl.BlockSpec((BM, H), xmap),
            pl.BlockSpec((1, H, I), wmap_g),
            pl.BlockSpec((1, H, I), wmap_u),
        ],
        out_specs=pl.BlockSpec((BM, I), omap),
    )
    return pl.pallas_call(
        _gu_body,
        grid_spec=grid_spec,
        out_shape=jax.ShapeDtypeStruct((PT, I), jnp.bfloat16),
    )(tile_expert, n_tiles, x_pad, w_gate_up, w_gate_up)


def _down_body(te_ref, nt_ref, a_ref, wd_ref, y_ref):
    t = pl.program_id(0)

    @pl.when(t < nt_ref[0])
    def _():
        y_ref[...] = jnp.dot(a_ref[...].astype(jnp.float32), wd_ref[0],
                             preferred_element_type=jnp.float32
                             ).astype(jnp.bfloat16)


def _grouped_down(act_pad, w_down, tile_expert, n_tiles):
    def amap(t, te, nt):
        return jnp.where(t < nt[0], t, nt[0] - 1), 0

    def wmap(t, te, nt):
        return te[jnp.where(t < nt[0], t, nt[0] - 1)], 0, 0

    def omap(t, te, nt):
        return jnp.where(t < nt[0], t, nt[0] - 1), 0

    grid_spec = pltpu.PrefetchScalarGridSpec(
        num_scalar_prefetch=2,
        grid=(NT,),
        in_specs=[
            pl.BlockSpec((BM, I), amap),
            pl.BlockSpec((1, I, H), wmap),
        ],
        out_specs=pl.BlockSpec((BM, H), omap),
    )
    return pl.pallas_call(
        _down_body,
        grid_spec=grid_spec,
        out_shape=jax.ShapeDtypeStruct((PT, H), jnp.bfloat16),
    )(tile_expert, n_tiles, act_pad, w_down)


def _shared_gu_body(x_ref, wg_ref, wu_ref, act_ref):
    g = jnp.dot(x_ref[...], wg_ref[...], preferred_element_type=jnp.float32)
    u = jnp.dot(x_ref[...], wu_ref[...], preferred_element_type=jnp.float32)
    act_ref[...] = (g * jax.nn.sigmoid(g) * u).astype(jnp.bfloat16)


def _shared_gate_up(x, ws_gate_up):
    nbn = ISH // BN_SH
    return pl.pallas_call(
        _shared_gu_body,
        grid=(T // BM, nbn),
        in_specs=[
            pl.BlockSpec((BM, H), lambda t, n: (t, 0)),
            pl.BlockSpec((H, BN_SH), lambda t, n: (0, n)),
            pl.BlockSpec((H, BN_SH), lambda t, n: (0, n + nbn)),
        ],
        out_specs=pl.BlockSpec((BM, BN_SH), lambda t, n: (t, n)),
        out_shape=jax.ShapeDtypeStruct((T, ISH), jnp.bfloat16),
    )(x, ws_gate_up, ws_gate_up)


def _combine_body(a_ref, wd_ref, y_ref, w_ref, out_ref):
    sh = jnp.dot(a_ref[...].astype(jnp.float32), wd_ref[...],
                 preferred_element_type=jnp.float32)
    w0 = w_ref[:, 0:1]
    w1 = w_ref[:, 1:2]
    routed = (w0 * y_ref[:, 0, :].astype(jnp.float32)
              + w1 * y_ref[:, 1, :].astype(jnp.float32))
    out_ref[...] = sh + RSF * routed


def _shared_down_combine(act_sh, ws_down, y_pairs, topk_w):
    nbh = H // BH
    return pl.pallas_call(
        _combine_body,
        grid=(T // BM, nbh),
        in_specs=[
            pl.BlockSpec((BM, ISH), lambda t, n: (t, 0)),
            pl.BlockSpec((ISH, BH), lambda t, n: (0, n)),
            pl.BlockSpec((BM, TOPK, BH), lambda t, n: (t, 0, n)),
            pl.BlockSpec((BM, TOPK), lambda t, n: (t, 0)),
        ],
        out_specs=pl.BlockSpec((BM, BH), lambda t, n: (t, n)),
        out_shape=jax.ShapeDtypeStruct((T, H), jnp.float32),
    )(act_sh, ws_down, y_pairs, topk_w)


_NEG = -1e30
GSZ = E // NG


def _routing_body(x_ref, gw_ref, bias_ref, idx_ref, w_ref):
    logits = jnp.dot(x_ref[...], gw_ref[...], preferred_element_type=jnp.float32)
    scores = jax.nn.sigmoid(logits)
    sfc = scores + bias_ref[...]
    rows = x_ref.shape[0]
    lane = lax.broadcasted_iota(jnp.int32, (rows, E), 1)
    grp = lane // GSZ

    gcol = []
    for g in range(NG):
        vals = jnp.where(grp == g, sfc, _NEG)
        m1 = jnp.max(vals, axis=1, keepdims=True)
        i1 = jnp.min(jnp.where(vals == m1, lane, E), axis=1, keepdims=True)
        m2 = jnp.max(jnp.where(lane == i1, _NEG, vals), axis=1, keepdims=True)
        gcol.append(m1 + m2)

    selmask = jnp.zeros((rows, E), jnp.bool_)
    for g in range(NG):
        rank = jnp.zeros((rows, 1), jnp.int32)
        for g2 in range(NG):
            if g2 == g:
                continue
            beats = gcol[g2] >= gcol[g] if g2 < g else gcol[g2] > gcol[g]
            rank = rank + beats.astype(jnp.int32)
        selmask = selmask | ((grp == g) & (rank < TKG))

    tmp = jnp.where(selmask, sfc, 0.0)
    m1 = jnp.max(tmp, axis=1, keepdims=True)
    i1 = jnp.min(jnp.where(tmp == m1, lane, E), axis=1, keepdims=True)
    tmp2 = jnp.where(lane == i1, _NEG, tmp)
    m2 = jnp.max(tmp2, axis=1, keepdims=True)
    i2 = jnp.min(jnp.where(tmp2 == m2, lane, E), axis=1, keepdims=True)
    w1 = jnp.sum(jnp.where(lane == i1, scores, 0.0), axis=1, keepdims=True)
    w2 = jnp.sum(jnp.where(lane == i2, scores, 0.0), axis=1, keepdims=True)
    s = w1 + w2 + 1e-20
    idx_ref[...] = jnp.concatenate([i1, i2], axis=1).astype(jnp.int32)
    w_ref[...] = jnp.concatenate([w1 / s, w2 / s], axis=1)


def _routing(x, gate_w, e_bias):
    idx, w = pl.pallas_call(
        _routing_body,
        grid=(T // BM,),
        in_specs=[
            pl.BlockSpec((BM, H), lambda t: (t, 0)),
            pl.BlockSpec((H, E), lambda t: (0, 0)),
            pl.BlockSpec((1, E), lambda t: (0, 0)),
        ],
        out_specs=[
            pl.BlockSpec((BM, TOPK), lambda t: (t, 0)),
            pl.BlockSpec((BM, TOPK), lambda t: (t, 0)),
        ],
        out_shape=[
            jax.ShapeDtypeStruct((T, TOPK), jnp.int32),
            jax.ShapeDtypeStruct((T, TOPK), jnp.float32),
        ],
    )(x, gate_w, e_bias.reshape(1, E))
    return idx, w


def _dispatch_meta(topk_idx):
    expert_ids = topk_idx.reshape(-1)
    order = jnp.argsort(expert_ids).astype(jnp.int32)
    sorted_e = expert_ids[order]
    counts = jnp.zeros((E,), jnp.int32).at[expert_ids].add(1)
    tiles_per = (counts + BM - 1) // BM
    tile_cum = jnp.cumsum(tiles_per)
    n_tiles = tile_cum[-1]
    pad_start = (tile_cum - tiles_per) * BM
    grp_start = jnp.cumsum(counts) - counts
    j = jnp.arange(P, dtype=jnp.int32)
    dest_row = pad_start[sorted_e] + (j - grp_start[sorted_e])
    row_src = (jnp.arange(PT, dtype=jnp.int32) % T).at[dest_row].set(
        order // TOPK)
    dest_pair = jnp.zeros((P,), jnp.int32).at[order].set(dest_row)
    dest_pair = dest_pair.reshape(T, TOPK)
    tile_expert = jnp.searchsorted(tile_cum, jnp.arange(NT), side="right")
    tile_expert = jnp.minimum(tile_expert, E - 1).astype(jnp.int32)
    n_tiles = jnp.reshape(n_tiles, (1,)).astype(jnp.int32)
    return row_src, dest_pair, tile_expert, n_tiles


def kernel(hidden_states, gate_w, e_bias, w_gate_up, w_down, ws_gate_up,
           ws_down):
    x = hidden_states

    topk_idx, topk_w = _routing(x, gate_w, e_bias)
    row_src, dest_pair, tile_expert, n_tiles = _dispatch_meta(topk_idx)

    x_bf = x.astype(jnp.bfloat16)
    x_i = lax.bitcast_convert_type(x_bf.reshape(T, H // 2, 2), jnp.int32)
    xp_i = _sc_row_gather(x_i, row_src, PT)
    x_pad = lax.bitcast_convert_type(xp_i, jnp.bfloat16).reshape(PT, H)

    act_pad = _grouped_gate_up(x_pad, w_gate_up, tile_expert, n_tiles)
    y_pad = _grouped_down(act_pad, w_down, tile_expert, n_tiles)

    y_i = lax.bitcast_convert_type(y_pad.reshape(PT, H // 2, 2), jnp.int32)
    yp_i = _sc_row_gather(y_i, dest_pair.reshape(-1), P)
    y_pairs = lax.bitcast_convert_type(yp_i, jnp.bfloat16).reshape(T, TOPK, H)

    act_sh = _shared_gate_up(x, ws_gate_up)
    out = _shared_down_combine(act_sh, ws_down, y_pairs, topk_w)
    return out


# speedup vs baseline: 4.9203x; 4.9203x over previous
import functools

import jax
import jax.numpy as jnp
from jax import lax
from jax.experimental import pallas as pl
from jax.experimental.pallas import tpu as pltpu
from jax.experimental.pallas import tpu_sc as plsc

T = 2048
H = 2048
E = 16
I = 1408
ISH = 2816
NG = 4
TKG = 2
TOPK = 2
RSF = 2.5

P = T * TOPK
BM = 256
NT = P // BM + E
PT = NT * BM

BN_SH = 1408
BH = 1024


NC_SC = 2
NS_SC = 16
NW_SC = NC_SC * NS_SC
GC = 16


def _sc_row_gather(table, idx, n_rows, used_rows=None):
    D = table.shape[1]
    per_w = n_rows // NW_SC
    n_chunks = per_w // GC
    idx2d = idx.reshape(-1, GC)
    mesh = plsc.VectorSubcoreMesh(core_axis_name="c", subcore_axis_name="s")

    @functools.partial(
        pl.kernel, mesh=mesh,
        out_type=jax.ShapeDtypeStruct((n_rows, D), table.dtype),
        scratch_types=[
            pltpu.VMEM((n_chunks, GC), jnp.int32),
            pltpu.VMEM((2, GC, D), table.dtype),
            pltpu.SemaphoreType.DMA,
            pltpu.SemaphoreType.DMA,
        ],
    )
    def gather_k(table_hbm, idx_hbm, out_hbm, idx_v, rows_v, sem0, sem1):
        wid = lax.axis_index("s") * NC_SC + lax.axis_index("c")
        base = wid * per_w
        pltpu.sync_copy(idx_hbm.at[pl.ds(wid * n_chunks, n_chunks)], idx_v)
        pltpu.async_copy(table_hbm.at[idx_v.at[0]], rows_v.at[0], sem0)

        def body(g, carry):
            i0 = g * 2
            pltpu.make_async_copy(
                table_hbm.at[idx_v.at[i0]], rows_v.at[0], sem0).wait()
            pltpu.async_copy(
                table_hbm.at[idx_v.at[i0 + 1]], rows_v.at[1], sem1)
            pltpu.sync_copy(
                rows_v.at[0],
                out_hbm.at[pl.ds(pl.multiple_of(base + i0 * GC, GC), GC)])
            pltpu.make_async_copy(
                table_hbm.at[idx_v.at[i0 + 1]], rows_v.at[1], sem1).wait()

            @pl.when(i0 + 2 < n_chunks)
            def _():
                pltpu.async_copy(
                    table_hbm.at[idx_v.at[i0 + 2]], rows_v.at[0], sem0)

            pltpu.sync_copy(
                rows_v.at[1],
                out_hbm.at[pl.ds(pl.multiple_of(base + (i0 + 1) * GC, GC),
                                 GC)])
            return carry

        lax.fori_loop(0, n_chunks // 2, body, 0)

    return gather_k(table, idx2d)


def _gu_body(te_ref, nt_ref, x_ref, wg_ref, wu_ref, act_ref):
    t = pl.program_id(0)

    @pl.when(t < nt_ref[0])
    def _():
        xb = x_ref[...].astype(jnp.float32)
        g = jnp.dot(xb, wg_ref[0], preferred_element_type=jnp.float32)
        u = jnp.dot(xb, wu_ref[0], preferred_element_type=jnp.float32)
        act_ref[...] = (g * jax.nn.sigmoid(g) * u).astype(jnp.bfloat16)


def _grouped_gate_up(x_pad, w_gate_up, tile_expert, n_tiles):
    def xmap(t, te, nt):
        return jnp.where(t < nt[0], t, nt[0] - 1), 0

    def wmap_g(t, te, nt):
        return te[jnp.where(t < nt[0], t, nt[0] - 1)], 0, 0

    def wmap_u(t, te, nt):
        return te[jnp.where(t < nt[0], t, nt[0] - 1)], 0, 1

    def omap(t, te, nt):
        return jnp.where(t < nt[0], t, nt[0] - 1), 0

    grid_spec = pltpu.PrefetchScalarGridSpec(
        num_scalar_prefetch=2,
        grid=(NT,),
        in_specs=[
            pl.BlockSpec((BM, H), xmap),
            pl.BlockSpec((1, H, I), wmap_g),
            pl.BlockSpec((1, H, I), wmap_u),
        ],
        out_specs=pl.BlockSpec((BM, I), omap),
    )
    return pl.pallas_call(
        _gu_body,
        grid_spec=grid_spec,
        out_shape=jax.ShapeDtypeStruct((PT, I), jnp.bfloat16),
    )(tile_expert, n_tiles, x_pad, w_gate_up, w_gate_up)


def _down_body(te_ref, nt_ref, a_ref, wd_ref, y_ref):
    t = pl.program_id(0)

    @pl.when(t < nt_ref[0])
    def _():
        y_ref[...] = jnp.dot(a_ref[...].astype(jnp.float32), wd_ref[0],
                             preferred_element_type=jnp.float32)


def _grouped_down(act_pad, w_down, tile_expert, n_tiles):
    def amap(t, te, nt):
        return jnp.where(t < nt[0], t, nt[0] - 1), 0

    def wmap(t, te, nt):
        return te[jnp.where(t < nt[0], t, nt[0] - 1)], 0, 0

    def omap(t, te, nt):
        return jnp.where(t < nt[0], t, nt[0] - 1), 0

    grid_spec = pltpu.PrefetchScalarGridSpec(
        num_scalar_prefetch=2,
        grid=(NT,),
        in_specs=[
            pl.BlockSpec((BM, I), amap),
            pl.BlockSpec((1, I, H), wmap),
        ],
        out_specs=pl.BlockSpec((BM, H), omap),
    )
    return pl.pallas_call(
        _down_body,
        grid_spec=grid_spec,
        out_shape=jax.ShapeDtypeStruct((PT, H), jnp.float32),
    )(tile_expert, n_tiles, act_pad, w_down)


def _shared_gu_body(x_ref, wg_ref, wu_ref, act_ref):
    g = jnp.dot(x_ref[...], wg_ref[...], preferred_element_type=jnp.float32)
    u = jnp.dot(x_ref[...], wu_ref[...], preferred_element_type=jnp.float32)
    act_ref[...] = (g * jax.nn.sigmoid(g) * u).astype(jnp.bfloat16)


def _shared_gate_up(x, ws_gate_up):
    nbn = ISH // BN_SH
    return pl.pallas_call(
        _shared_gu_body,
        grid=(T // BM, nbn),
        in_specs=[
            pl.BlockSpec((BM, H), lambda t, n: (t, 0)),
            pl.BlockSpec((H, BN_SH), lambda t, n: (0, n)),
            pl.BlockSpec((H, BN_SH), lambda t, n: (0, n + nbn)),
        ],
        out_specs=pl.BlockSpec((BM, BN_SH), lambda t, n: (t, n)),
        out_shape=jax.ShapeDtypeStruct((T, ISH), jnp.bfloat16),
    )(x, ws_gate_up, ws_gate_up)


def _combine_body(a_ref, wd_ref, y_ref, w_ref, out_ref):
    sh = jnp.dot(a_ref[...].astype(jnp.float32), wd_ref[...],
                 preferred_element_type=jnp.float32)
    w0 = w_ref[:, 0:1]
    w1 = w_ref[:, 1:2]
    routed = (w0 * y_ref[:, 0, :].astype(jnp.float32)
              + w1 * y_ref[:, 1, :].astype(jnp.float32))
    out_ref[...] = sh + RSF * routed


def _shared_down_combine(act_sh, ws_down, y_pairs, topk_w):
    nbh = H // BH
    return pl.pallas_call(
        _combine_body,
        grid=(T // BM, nbh),
        in_specs=[
            pl.BlockSpec((BM, ISH), lambda t, n: (t, 0)),
            pl.BlockSpec((ISH, BH), lambda t, n: (0, n)),
            pl.BlockSpec((BM, TOPK, BH), lambda t, n: (t, 0, n)),
            pl.BlockSpec((BM, TOPK), lambda t, n: (t, 0)),
        ],
        out_specs=pl.BlockSpec((BM, BH), lambda t, n: (t, n)),
        out_shape=jax.ShapeDtypeStruct((T, H), jnp.float32),
    )(act_sh, ws_down, y_pairs, topk_w)


_NEG = -1e30
GSZ = E // NG


def _routing_body(x_ref, gw_ref, bias_ref, idx_ref, w_ref):
    logits = jnp.dot(x_ref[...], gw_ref[...], preferred_element_type=jnp.float32)
    scores = jax.nn.sigmoid(logits)
    sfc = scores + bias_ref[...]
    rows = x_ref.shape[0]
    lane = lax.broadcasted_iota(jnp.int32, (rows, E), 1)
    grp = lane // GSZ

    gcol = []
    for g in range(NG):
        vals = jnp.where(grp == g, sfc, _NEG)
        m1 = jnp.max(vals, axis=1, keepdims=True)
        i1 = jnp.min(jnp.where(vals == m1, lane, E), axis=1, keepdims=True)
        m2 = jnp.max(jnp.where(lane == i1, _NEG, vals), axis=1, keepdims=True)
        gcol.append(m1 + m2)

    selmask = jnp.zeros((rows, E), jnp.bool_)
    for g in range(NG):
        rank = jnp.zeros((rows, 1), jnp.int32)
        for g2 in range(NG):
            if g2 == g:
                continue
            beats = gcol[g2] >= gcol[g] if g2 < g else gcol[g2] > gcol[g]
            rank = rank + beats.astype(jnp.int32)
        selmask = selmask | ((grp == g) & (rank < TKG))

    tmp = jnp.where(selmask, sfc, 0.0)
    m1 = jnp.max(tmp, axis=1, keepdims=True)
    i1 = jnp.min(jnp.where(tmp == m1, lane, E), axis=1, keepdims=True)
    tmp2 = jnp.where(lane == i1, _NEG, tmp)
    m2 = jnp.max(tmp2, axis=1, keepdims=True)
    i2 = jnp.min(jnp.where(tmp2 == m2, lane, E), axis=1, keepdims=True)
    w1 = jnp.sum(jnp.where(lane == i1, scores, 0.0), axis=1, keepdims=True)
    w2 = jnp.sum(jnp.where(lane == i2, scores, 0.0), axis=1, keepdims=True)
    s = w1 + w2 + 1e-20
    idx_ref[...] = jnp.concatenate([i1, i2], axis=1).astype(jnp.int32)
    w_ref[...] = jnp.concatenate([w1 / s, w2 / s], axis=1)


def _routing(x, gate_w, e_bias):
    idx, w = pl.pallas_call(
        _routing_body,
        grid=(T // BM,),
        in_specs=[
            pl.BlockSpec((BM, H), lambda t: (t, 0)),
            pl.BlockSpec((H, E), lambda t: (0, 0)),
            pl.BlockSpec((1, E), lambda t: (0, 0)),
        ],
        out_specs=[
            pl.BlockSpec((BM, TOPK), lambda t: (t, 0)),
            pl.BlockSpec((BM, TOPK), lambda t: (t, 0)),
        ],
        out_shape=[
            jax.ShapeDtypeStruct((T, TOPK), jnp.int32),
            jax.ShapeDtypeStruct((T, TOPK), jnp.float32),
        ],
    )(x, gate_w, e_bias.reshape(1, E))
    return idx, w


def _dispatch_meta(topk_idx):
    expert_ids = topk_idx.reshape(-1)
    order = jnp.argsort(expert_ids).astype(jnp.int32)
    sorted_e = expert_ids[order]
    counts = jnp.zeros((E,), jnp.int32).at[expert_ids].add(1)
    tiles_per = (counts + BM - 1) // BM
    tile_cum = jnp.cumsum(tiles_per)
    n_tiles = tile_cum[-1]
    pad_start = (tile_cum - tiles_per) * BM
    grp_start = jnp.cumsum(counts) - counts
    j = jnp.arange(P, dtype=jnp.int32)
    dest_row = pad_start[sorted_e] + (j - grp_start[sorted_e])
    row_src = (jnp.arange(PT, dtype=jnp.int32) % T).at[dest_row].set(
        order // TOPK)
    dest_pair = jnp.zeros((P,), jnp.int32).at[order].set(dest_row)
    dest_pair = dest_pair.reshape(T, TOPK)
    tile_expert = jnp.searchsorted(tile_cum, jnp.arange(NT), side="right")
    tile_expert = jnp.minimum(tile_expert, E - 1).astype(jnp.int32)
    n_tiles = jnp.reshape(n_tiles, (1,)).astype(jnp.int32)
    return row_src, dest_pair, tile_expert, n_tiles


def kernel(hidden_states, gate_w, e_bias, w_gate_up, w_down, ws_gate_up,
           ws_down):
    x = hidden_states

    topk_idx, topk_w = _routing(x, gate_w, e_bias)
    row_src, dest_pair, tile_expert, n_tiles = _dispatch_meta(topk_idx)

    x_pad = _sc_row_gather(x, row_src, PT)

    act_pad = _grouped_gate_up(x_pad, w_gate_up, tile_expert, n_tiles)
    y_pad = _grouped_down(act_pad, w_down, tile_expert, n_tiles)

    y_pairs = _sc_row_gather(y_pad, dest_pair.reshape(-1), P)
    y_pairs = y_pairs.reshape(T, TOPK, H)

    act_sh = _shared_gate_up(x, ws_gate_up)
    out = _shared_down_combine(act_sh, ws_down, y_pairs, topk_w)
    return out


# speedup vs baseline: 4.9238x; 1.0007x over previous
import functools

import jax
import jax.numpy as jnp
from jax import lax
from jax.experimental import pallas as pl
from jax.experimental.pallas import tpu as pltpu
from jax.experimental.pallas import tpu_sc as plsc

T = 2048
H = 2048
E = 16
I = 1408
ISH = 2816
NG = 4
TKG = 2
TOPK = 2
RSF = 2.5

P = T * TOPK
BM = 256
NT = P // BM + E
PT = NT * BM

BN_SH = 1408
BH = 1024


NC_SC = 2
NS_SC = 16
NW_SC = NC_SC * NS_SC
GC = 16


def _sc_row_gather(table, idx, n_rows, used_rows=None):
    D = table.shape[1]
    per_w = n_rows // NW_SC
    n_chunks = per_w // GC
    idx2d = idx.reshape(-1, GC)
    mesh = plsc.VectorSubcoreMesh(core_axis_name="c", subcore_axis_name="s")

    @functools.partial(
        pl.kernel, mesh=mesh,
        out_type=jax.ShapeDtypeStruct((n_rows, D), table.dtype),
        scratch_types=[
            pltpu.VMEM((n_chunks, GC), jnp.int32),
            pltpu.VMEM((2, GC, D), table.dtype),
            pltpu.SemaphoreType.DMA,
            pltpu.SemaphoreType.DMA,
        ],
    )
    def gather_k(table_hbm, idx_hbm, out_hbm, idx_v, rows_v, sem0, sem1):
        wid = lax.axis_index("s") * NC_SC + lax.axis_index("c")
        base = wid * per_w
        pltpu.sync_copy(idx_hbm.at[pl.ds(wid * n_chunks, n_chunks)], idx_v)
        pltpu.async_copy(table_hbm.at[idx_v.at[0]], rows_v.at[0], sem0)

        def body(g, carry):
            i0 = g * 2
            pltpu.make_async_copy(
                table_hbm.at[idx_v.at[i0]], rows_v.at[0], sem0).wait()
            pltpu.async_copy(
                table_hbm.at[idx_v.at[i0 + 1]], rows_v.at[1], sem1)
            pltpu.sync_copy(
                rows_v.at[0],
                out_hbm.at[pl.ds(pl.multiple_of(base + i0 * GC, GC), GC)])
            pltpu.make_async_copy(
                table_hbm.at[idx_v.at[i0 + 1]], rows_v.at[1], sem1).wait()

            @pl.when(i0 + 2 < n_chunks)
            def _():
                pltpu.async_copy(
                    table_hbm.at[idx_v.at[i0 + 2]], rows_v.at[0], sem0)

            pltpu.sync_copy(
                rows_v.at[1],
                out_hbm.at[pl.ds(pl.multiple_of(base + (i0 + 1) * GC, GC),
                                 GC)])
            return carry

        lax.fori_loop(0, n_chunks // 2, body, 0)

    return gather_k(table, idx2d)


def _gu_body(te_ref, tr_ref, nt_ref, x_ref, wg_ref, wu_ref, act_ref):
    t = pl.program_id(0)

    @pl.when(t < nt_ref[0])
    def _():
        xb = x_ref[...].astype(jnp.float32)
        g = jnp.dot(xb, wg_ref[0], preferred_element_type=jnp.float32)
        u = jnp.dot(xb, wu_ref[0], preferred_element_type=jnp.float32)
        act_ref[...] = (g * jax.nn.sigmoid(g) * u).astype(jnp.bfloat16)


def _grouped_gate_up(x_pad, w_gate_up, tile_expert, tile_row, n_tiles):
    grid_spec = pltpu.PrefetchScalarGridSpec(
        num_scalar_prefetch=3,
        grid=(NT,),
        in_specs=[
            pl.BlockSpec((BM, H), lambda t, te, tr, nt: (tr[t], 0)),
            pl.BlockSpec((1, H, I), lambda t, te, tr, nt: (te[t], 0, 0)),
            pl.BlockSpec((1, H, I), lambda t, te, tr, nt: (te[t], 0, 1)),
        ],
        out_specs=pl.BlockSpec((BM, I), lambda t, te, tr, nt: (tr[t], 0)),
    )
    return pl.pallas_call(
        _gu_body,
        grid_spec=grid_spec,
        out_shape=jax.ShapeDtypeStruct((PT, I), jnp.bfloat16),
    )(tile_expert, tile_row, n_tiles, x_pad, w_gate_up, w_gate_up)


def _down_body(te_ref, tr_ref, nt_ref, a_ref, wd_ref, y_ref):
    t = pl.program_id(0)

    @pl.when(t < nt_ref[0])
    def _():
        y_ref[...] = jnp.dot(a_ref[...].astype(jnp.float32), wd_ref[0],
                             preferred_element_type=jnp.float32)


def _grouped_down(act_pad, w_down, tile_expert, tile_row, n_tiles):
    grid_spec = pltpu.PrefetchScalarGridSpec(
        num_scalar_prefetch=3,
        grid=(NT,),
        in_specs=[
            pl.BlockSpec((BM, I), lambda t, te, tr, nt: (tr[t], 0)),
            pl.BlockSpec((1, I, H), lambda t, te, tr, nt: (te[t], 0, 0)),
        ],
        out_specs=pl.BlockSpec((BM, H), lambda t, te, tr, nt: (tr[t], 0)),
    )
    return pl.pallas_call(
        _down_body,
        grid_spec=grid_spec,
        out_shape=jax.ShapeDtypeStruct((PT, H), jnp.float32),
    )(tile_expert, tile_row, n_tiles, act_pad, w_down)


def _shared_gu_body(x_ref, wg_ref, wu_ref, act_ref):
    g = jnp.dot(x_ref[...], wg_ref[...], preferred_element_type=jnp.float32)
    u = jnp.dot(x_ref[...], wu_ref[...], preferred_element_type=jnp.float32)
    act_ref[...] = (g * jax.nn.sigmoid(g) * u).astype(jnp.bfloat16)


def _shared_gate_up(x, ws_gate_up):
    nbn = ISH // BN_SH
    return pl.pallas_call(
        _shared_gu_body,
        grid=(T // BM, nbn),
        in_specs=[
            pl.BlockSpec((BM, H), lambda t, n: (t, 0)),
            pl.BlockSpec((H, BN_SH), lambda t, n: (0, n)),
            pl.BlockSpec((H, BN_SH), lambda t, n: (0, n + nbn)),
        ],
        out_specs=pl.BlockSpec((BM, BN_SH), lambda t, n: (t, n)),
        out_shape=jax.ShapeDtypeStruct((T, ISH), jnp.bfloat16),
    )(x, ws_gate_up, ws_gate_up)


def _combine_body(a_ref, wd_ref, y_ref, w_ref, out_ref):
    sh = jnp.dot(a_ref[...].astype(jnp.float32), wd_ref[...],
                 preferred_element_type=jnp.float32)
    w0 = w_ref[:, 0:1]
    w1 = w_ref[:, 1:2]
    routed = (w0 * y_ref[:, 0, :].astype(jnp.float32)
              + w1 * y_ref[:, 1, :].astype(jnp.float32))
    out_ref[...] = sh + RSF * routed


def _shared_down_combine(act_sh, ws_down, y_pairs, topk_w):
    nbh = H // BH
    return pl.pallas_call(
        _combine_body,
        grid=(T // BM, nbh),
        in_specs=[
            pl.BlockSpec((BM, ISH), lambda t, n: (t, 0)),
            pl.BlockSpec((ISH, BH), lambda t, n: (0, n)),
            pl.BlockSpec((BM, TOPK, BH), lambda t, n: (t, 0, n)),
            pl.BlockSpec((BM, TOPK), lambda t, n: (t, 0)),
        ],
        out_specs=pl.BlockSpec((BM, BH), lambda t, n: (t, n)),
        out_shape=jax.ShapeDtypeStruct((T, H), jnp.float32),
    )(act_sh, ws_down, y_pairs, topk_w)


_NEG = -1e30
GSZ = E // NG


def _routing_body(x_ref, gw_ref, bias_ref, idx_ref, w_ref):
    logits = jnp.dot(x_ref[...], gw_ref[...], preferred_element_type=jnp.float32)
    scores = jax.nn.sigmoid(logits)
    sfc = scores + bias_ref[...]
    rows = x_ref.shape[0]
    lane = lax.broadcasted_iota(jnp.int32, (rows, E), 1)
    grp = lane // GSZ

    gcol = []
    for g in range(NG):
        vals = jnp.where(grp == g, sfc, _NEG)
        m1 = jnp.max(vals, axis=1, keepdims=True)
        i1 = jnp.min(jnp.where(vals == m1, lane, E), axis=1, keepdims=True)
        m2 = jnp.max(jnp.where(lane == i1, _NEG, vals), axis=1, keepdims=True)
        gcol.append(m1 + m2)

    selmask = jnp.zeros((rows, E), jnp.bool_)
    for g in range(NG):
        rank = jnp.zeros((rows, 1), jnp.int32)
        for g2 in range(NG):
            if g2 == g:
                continue
            beats = gcol[g2] >= gcol[g] if g2 < g else gcol[g2] > gcol[g]
            rank = rank + beats.astype(jnp.int32)
        selmask = selmask | ((grp == g) & (rank < TKG))

    tmp = jnp.where(selmask, sfc, 0.0)
    m1 = jnp.max(tmp, axis=1, keepdims=True)
    i1 = jnp.min(jnp.where(tmp == m1, lane, E), axis=1, keepdims=True)
    tmp2 = jnp.where(lane == i1, _NEG, tmp)
    m2 = jnp.max(tmp2, axis=1, keepdims=True)
    i2 = jnp.min(jnp.where(tmp2 == m2, lane, E), axis=1, keepdims=True)
    w1 = jnp.sum(jnp.where(lane == i1, scores, 0.0), axis=1, keepdims=True)
    w2 = jnp.sum(jnp.where(lane == i2, scores, 0.0), axis=1, keepdims=True)
    s = w1 + w2 + 1e-20
    idx_ref[...] = jnp.concatenate([i1, i2], axis=1).astype(jnp.int32)
    w_ref[...] = jnp.concatenate([w1 / s, w2 / s], axis=1)


def _routing(x, gate_w, e_bias):
    idx, w = pl.pallas_call(
        _routing_body,
        grid=(T // BM,),
        in_specs=[
            pl.BlockSpec((BM, H), lambda t: (t, 0)),
            pl.BlockSpec((H, E), lambda t: (0, 0)),
            pl.BlockSpec((1, E), lambda t: (0, 0)),
        ],
        out_specs=[
            pl.BlockSpec((BM, TOPK), lambda t: (t, 0)),
            pl.BlockSpec((BM, TOPK), lambda t: (t, 0)),
        ],
        out_shape=[
            jax.ShapeDtypeStruct((T, TOPK), jnp.int32),
            jax.ShapeDtypeStruct((T, TOPK), jnp.float32),
        ],
    )(x, gate_w, e_bias.reshape(1, E))
    return idx, w


def _dispatch_meta(topk_idx):
    expert_ids = topk_idx.reshape(-1)
    order = jnp.argsort(expert_ids).astype(jnp.int32)
    sorted_e = expert_ids[order]
    counts = jnp.zeros((E,), jnp.int32).at[expert_ids].add(1)
    tiles_per = (counts + BM - 1) // BM
    tile_cum = jnp.cumsum(tiles_per)
    n_tiles = tile_cum[-1]
    pad_start = (tile_cum - tiles_per) * BM
    grp_start = jnp.cumsum(counts) - counts
    j = jnp.arange(P, dtype=jnp.int32)
    dest_row = pad_start[sorted_e] + (j - grp_start[sorted_e])
    row_src = (jnp.arange(PT, dtype=jnp.int32) % T).at[dest_row].set(
        order // TOPK)
    dest_pair = jnp.zeros((P,), jnp.int32).at[order].set(dest_row)
    dest_pair = dest_pair.reshape(T, TOPK)
    tile_row = jnp.minimum(jnp.arange(NT, dtype=jnp.int32), n_tiles - 1)
    tile_expert = jnp.searchsorted(tile_cum, jnp.arange(NT), side="right")
    tile_expert = jnp.minimum(tile_expert, E - 1).astype(jnp.int32)[tile_row]
    n_tiles = jnp.reshape(n_tiles, (1,)).astype(jnp.int32)
    return row_src, dest_pair, tile_expert, tile_row, n_tiles


def kernel(hidden_states, gate_w, e_bias, w_gate_up, w_down, ws_gate_up,
           ws_down):
    x = hidden_states

    topk_idx, topk_w = _routing(x, gate_w, e_bias)
    row_src, dest_pair, tile_expert, tile_row, n_tiles = _dispatch_meta(
        topk_idx)

    x_pad = _sc_row_gather(x, row_src, PT)

    act_pad = _grouped_gate_up(x_pad, w_gate_up, tile_expert, tile_row,
                               n_tiles)
    y_pad = _grouped_down(act_pad, w_down, tile_expert, tile_row, n_tiles)

    y_pairs = _sc_row_gather(y_pad, dest_pair.reshape(-1), P)
    y_pairs = y_pairs.reshape(T, TOPK, H)

    act_sh = _shared_gate_up(x, ws_gate_up)
    out = _shared_down_combine(act_sh, ws_down, y_pairs, topk_w)
    return out
